# initial kernel scaffold (unmeasured)
import jax
import jax.numpy as jnp
import numpy as np
from jax import lax
from jax.experimental import pallas as pl
from jax.experimental.pallas import tpu as pltpu

N = 16
RH = 8
LH = 7

_RING = np.array([0, 4, 8, 12, 15, 11, 7, 3, 2, 6, 10, 14, 13, 9, 5, 1])
_POS = np.argsort(_RING)

_M_RIGHT, _M_LEFT, _M_SR, _M_RR, _M_SL, _M_RL = 0, 1, 2, 10, 18, 25


def kernel(x, w_mat):
    x16 = x.astype(jnp.bfloat16)
    w16 = w_mat.astype(jnp.bfloat16)
    m, kk = x16.shape
    _, n = w16.shape

    ring = jnp.asarray(_RING, jnp.int32)
    pos = jnp.asarray(_POS, jnp.int32)
    r = pos[lax.axis_index("i")]
    ids = [ring[(r + 1) % N], ring[(r - 1) % N]]
    ids += [ring[(r - h) % N] for h in range(RH)]
    ids += [ring[(r - h - 1) % N] for h in range(RH)]
    ids += [ring[(r + h) % N] for h in range(LH)]
    ids += [ring[(r + h + 1) % N] for h in range(LH)]
    meta = jnp.stack(ids).astype(jnp.int32)

    def body(meta_ref, x_ref, w_ref, out_ref, xg, wg, ssx, ssw, rsx, rsw, lsem):
        right = meta_ref[_M_RIGHT]
        left = meta_ref[_M_LEFT]

        barrier = pltpu.get_barrier_semaphore()
        for nbr in (right, left):
            pl.semaphore_signal(barrier, inc=1, device_id=(nbr,),
                                device_id_type=pl.DeviceIdType.MESH)
        pl.semaphore_wait(barrier, 2)

        def rdma(src, dst, send_sem, recv_sem, dev):
            return pltpu.make_async_remote_copy(
                src_ref=src, dst_ref=dst, send_sem=send_sem,
                recv_sem=recv_sem, device_id=(dev,),
                device_id_type=pl.DeviceIdType.MESH)

        my_slot = meta_ref[_M_SR]
        cpx = pltpu.make_async_copy(x_ref, xg.at[my_slot], lsem.at[0])
        cpw = pltpu.make_async_copy(w_ref, wg.at[my_slot], lsem.at[1])
        cpx.start()
        cpw.start()

        sends = []
        for h in range(RH):
            sor = meta_ref[_M_SR + h]
            sx = rdma(x_ref if h == 0 else xg.at[sor], xg.at[sor],
                      ssx.at[0, h], rsx.at[0, h], right)
            sw = rdma(w_ref if h == 0 else wg.at[sor], wg.at[sor],
                      ssw.at[0, h], rsw.at[0, h], right)
            sx.start()
            sw.start()
            sends += [sx, sw]
            if h < LH:
                sol = meta_ref[_M_SL + h]
                lx = rdma(x_ref if h == 0 else xg.at[sol], xg.at[sol],
                          ssx.at[1, h], rsx.at[1, h], left)
                lw = rdma(w_ref if h == 0 else wg.at[sol], wg.at[sol],
                          ssw.at[1, h], rsw.at[1, h], left)
                lx.start()
                lw.start()
                sends += [lx, lw]

            ror = meta_ref[_M_RR + h]
            rdma(x_ref, xg.at[ror], ssx.at[0, h], rsx.at[0, h], left).wait_recv()
            rdma(w_ref, wg.at[ror], ssw.at[0, h], rsw.at[0, h], left).wait_recv()
            if h < LH:
                rol = meta_ref[_M_RL + h]
                rdma(x_ref, xg.at[rol], ssx.at[1, h], rsx.at[1, h],
                     right).wait_recv()
                rdma(w_ref, wg.at[rol], ssw.at[1, h], rsw.at[1, h],
                     right).wait_recv()

        for s in sends:
            s.wait_send()
        cpx.wait()
        cpw.wait()

        acc = jax.lax.dot_general(
            xg[...], wg[...],
            dimension_numbers=(((0, 2), (0, 1)), ((), ())),
            preferred_element_type=jnp.float32)
        out_ref[...] = jnp.maximum(acc, 0.0)

    return pl.pallas_call(
        body,
        out_shape=jax.ShapeDtypeStruct((m, n), jnp.float32),
        in_specs=[
            pl.BlockSpec(memory_space=pltpu.SMEM),
            pl.BlockSpec(memory_space=pltpu.VMEM),
            pl.BlockSpec(memory_space=pltpu.VMEM),
        ],
        out_specs=pl.BlockSpec(memory_space=pltpu.VMEM),
        scratch_shapes=[
            pltpu.VMEM((N, m, kk), jnp.bfloat16),
            pltpu.VMEM((N, kk, n), jnp.bfloat16),
            pltpu.SemaphoreType.DMA((2, RH)),
            pltpu.SemaphoreType.DMA((2, RH)),
            pltpu.SemaphoreType.DMA((2, RH)),
            pltpu.SemaphoreType.DMA((2, RH)),
            pltpu.SemaphoreType.DMA((2,)),
        ],
        compiler_params=pltpu.CompilerParams(collective_id=0),
    )(meta, x16, w16)


# baseline (device time: 90577 ns/iter reference)
import jax
import jax.numpy as jnp
import numpy as np
from jax import lax
from jax.experimental import pallas as pl
from jax.experimental.pallas import tpu as pltpu

N = 16
RH = 8
LH = 7

_RING = np.array([0, 4, 8, 12, 15, 11, 7, 3, 2, 6, 10, 14, 13, 9, 5, 1])
_POS = np.argsort(_RING)

_M_RIGHT, _M_LEFT, _M_SR, _M_RR, _M_SL, _M_RL = 0, 1, 2, 10, 18, 25


def kernel(x, w_mat):
    x16 = x.astype(jnp.bfloat16)
    w16 = w_mat.astype(jnp.bfloat16)
    m, kk = x16.shape
    _, n = w16.shape

    ring = jnp.asarray(_RING, jnp.int32)
    pos = jnp.asarray(_POS, jnp.int32)
    r = pos[lax.axis_index("i")]
    ids = [ring[(r + 1) % N], ring[(r - 1) % N]]
    ids += [ring[(r - h) % N] for h in range(RH)]
    ids += [ring[(r - h - 1) % N] for h in range(RH)]
    ids += [ring[(r + h) % N] for h in range(LH)]
    ids += [ring[(r + h + 1) % N] for h in range(LH)]
    meta = jnp.stack(ids).astype(jnp.int32)

    def body(meta_ref, x_ref, w_ref, out_ref, xg, wg, ssx, ssw, rsx, rsw, lsem):
        right = meta_ref[_M_RIGHT]
        left = meta_ref[_M_LEFT]

        barrier = pltpu.get_barrier_semaphore()
        for nbr in (right, left):
            pl.semaphore_signal(barrier, inc=1, device_id=(nbr,),
                                device_id_type=pl.DeviceIdType.MESH)
        pl.semaphore_wait(barrier, 2)

        def rdma(src, dst, send_sem, recv_sem, dev):
            return pltpu.make_async_remote_copy(
                src_ref=src, dst_ref=dst, send_sem=send_sem,
                recv_sem=recv_sem, device_id=(dev,),
                device_id_type=pl.DeviceIdType.MESH)

        my_slot = meta_ref[_M_SR]
        cpx = pltpu.make_async_copy(x_ref, xg.at[my_slot], lsem.at[0])
        cpw = pltpu.make_async_copy(w_ref, wg.at[my_slot], lsem.at[1])
        cpx.start()
        cpw.start()

        sends = []
        for h in range(RH):
            sor = meta_ref[_M_SR + h]
            sx = rdma(x_ref if h == 0 else xg.at[sor], xg.at[sor],
                      ssx.at[0, h], rsx.at[0, h], right)
            sw = rdma(w_ref if h == 0 else wg.at[sor], wg.at[sor],
                      ssw.at[0, h], rsw.at[0, h], right)
            sx.start()
            sw.start()
            sends += [sx, sw]
            if h < LH:
                sol = meta_ref[_M_SL + h]
                lx = rdma(x_ref if h == 0 else xg.at[sol], xg.at[sol],
                          ssx.at[1, h], rsx.at[1, h], left)
                lw = rdma(w_ref if h == 0 else wg.at[sol], wg.at[sol],
                          ssw.at[1, h], rsw.at[1, h], left)
                lx.start()
                lw.start()
                sends += [lx, lw]

            ror = meta_ref[_M_RR + h]
            rdma(x_ref, xg.at[ror], ssx.at[0, h], rsx.at[0, h], left).wait_recv()
            rdma(w_ref, wg.at[ror], ssw.at[0, h], rsw.at[0, h], left).wait_recv()
            if h < LH:
                rol = meta_ref[_M_RL + h]
                rdma(x_ref, xg.at[rol], ssx.at[1, h], rsx.at[1, h],
                     right).wait_recv()
                rdma(w_ref, wg.at[rol], ssw.at[1, h], rsw.at[1, h],
                     right).wait_recv()

        for s in sends:
            s.wait_send()
        cpx.wait()
        cpw.wait()

        acc = jnp.dot(xg[0], wg[0], preferred_element_type=jnp.float32)
        for c in range(1, N):
            acc += jnp.dot(xg[c], wg[c], preferred_element_type=jnp.float32)
        out_ref[...] = jnp.maximum(acc, 0.0)

    return pl.pallas_call(
        body,
        out_shape=jax.ShapeDtypeStruct((m, n), jnp.float32),
        in_specs=[
            pl.BlockSpec(memory_space=pltpu.SMEM),
            pl.BlockSpec(memory_space=pltpu.VMEM),
            pl.BlockSpec(memory_space=pltpu.VMEM),
        ],
        out_specs=pl.BlockSpec(memory_space=pltpu.VMEM),
        scratch_shapes=[
            pltpu.VMEM((N, m, kk), jnp.bfloat16),
            pltpu.VMEM((N, kk, n), jnp.bfloat16),
            pltpu.SemaphoreType.DMA((2, RH)),
            pltpu.SemaphoreType.DMA((2, RH)),
            pltpu.SemaphoreType.DMA((2, RH)),
            pltpu.SemaphoreType.DMA((2, RH)),
            pltpu.SemaphoreType.DMA((2,)),
        ],
        compiler_params=pltpu.CompilerParams(collective_id=0),
    )(meta, x16, w16)


# device time: 53910 ns/iter; 1.6802x vs baseline; 1.6802x over previous
import jax
import jax.numpy as jnp
import numpy as np
from jax import lax
from jax.experimental import pallas as pl
from jax.experimental.pallas import tpu as pltpu

N = 16
RH = 8
LH = 7

_RING = np.array([0, 4, 8, 12, 15, 11, 7, 3, 2, 6, 10, 14, 13, 9, 5, 1])
_POS = np.argsort(_RING)

_M_RIGHT, _M_LEFT, _M_OWN, _M_RR, _M_RL = 0, 1, 2, 3, 11
_TBL = np.zeros((N, 18), np.int32)
for _l in range(N):
    _r = int(_POS[_l])
    _TBL[_l, _M_RIGHT] = _RING[(_r + 1) % N]
    _TBL[_l, _M_LEFT] = _RING[(_r - 1) % N]
    _TBL[_l, _M_OWN] = _l
    for _h in range(RH):
        _TBL[_l, _M_RR + _h] = _RING[(_r - _h - 1) % N]
    for _h in range(LH):
        _TBL[_l, _M_RL + _h] = _RING[(_r + _h + 1) % N]


def kernel(x, w_mat):
    xt16 = x.T.astype(jnp.bfloat16)
    w16 = w_mat.astype(jnp.bfloat16)
    kk, m = xt16.shape
    _, n = w16.shape

    def body(tbl_ref, xt_ref, w_ref, out_ref, xg, wg, ssx, ssw, rsx, rsw):
        my = lax.axis_index("i")
        right = tbl_ref[my, _M_RIGHT]
        left = tbl_ref[my, _M_LEFT]

        barrier = pltpu.get_barrier_semaphore()
        for nbr in (right, left):
            pl.semaphore_signal(barrier, inc=1, device_id=(nbr,),
                                device_id_type=pl.DeviceIdType.MESH)
        pl.semaphore_wait(barrier, 2)

        def rdma(src, dst, send_sem, recv_sem, dev):
            return pltpu.make_async_remote_copy(
                src_ref=src, dst_ref=dst, send_sem=send_sem,
                recv_sem=recv_sem, device_id=(dev,),
                device_id_type=pl.DeviceIdType.MESH)

        def gemm(xt_c, w_c):
            return lax.dot_general(
                xt_c, w_c, dimension_numbers=(((0,), (0,)), ((), ())),
                preferred_element_type=jnp.float32)

        own = tbl_ref[my, _M_OWN]
        sends = []
        for d, nbr in ((0, right), (1, left)):
            sx = rdma(xt_ref, xg.at[own], ssx.at[d, 0], rsx.at[d, 0], nbr)
            sw = rdma(w_ref, wg.at[own], ssw.at[d, 0], rsw.at[d, 0], nbr)
            sx.start()
            sw.start()
            sends += [sx, sw]

        acc = gemm(xt_ref[...], w_ref[...])
        for h in range(RH):
            ror = tbl_ref[my, _M_RR + h]
            rdma(xt_ref, xg.at[ror], ssx.at[0, h], rsx.at[0, h],
                 left).wait_recv()
            rdma(w_ref, wg.at[ror], ssw.at[0, h], rsw.at[0, h],
                 left).wait_recv()
            if h < LH:
                rol = tbl_ref[my, _M_RL + h]
                rdma(xt_ref, xg.at[rol], ssx.at[1, h], rsx.at[1, h],
                     right).wait_recv()
                rdma(w_ref, wg.at[rol], ssw.at[1, h], rsw.at[1, h],
                     right).wait_recv()

            if h + 1 < RH:
                sx = rdma(xg.at[ror], xg.at[ror], ssx.at[0, h + 1],
                          rsx.at[0, h + 1], right)
                sw = rdma(wg.at[ror], wg.at[ror], ssw.at[0, h + 1],
                          rsw.at[0, h + 1], right)
                sx.start()
                sw.start()
                sends += [sx, sw]
            if h + 1 < LH:
                lx = rdma(xg.at[rol], xg.at[rol], ssx.at[1, h + 1],
                          rsx.at[1, h + 1], left)
                lw = rdma(wg.at[rol], wg.at[rol], ssw.at[1, h + 1],
                          rsw.at[1, h + 1], left)
                lx.start()
                lw.start()
                sends += [lx, lw]

            acc += gemm(xg[ror], wg[ror])
            if h < LH:
                acc += gemm(xg[rol], wg[rol])

        out_ref[...] = jnp.maximum(acc, 0.0)
        for s in sends:
            s.wait_send()

    return pl.pallas_call(
        body,
        out_shape=jax.ShapeDtypeStruct((m, n), jnp.float32),
        in_specs=[
            pl.BlockSpec(memory_space=pltpu.SMEM),
            pl.BlockSpec(memory_space=pltpu.VMEM),
            pl.BlockSpec(memory_space=pltpu.VMEM),
        ],
        out_specs=pl.BlockSpec(memory_space=pltpu.VMEM),
        scratch_shapes=[
            pltpu.VMEM((N, kk, m), jnp.bfloat16),
            pltpu.VMEM((N, kk, n), jnp.bfloat16),
            pltpu.SemaphoreType.DMA((2, RH)),
            pltpu.SemaphoreType.DMA((2, RH)),
            pltpu.SemaphoreType.DMA((2, RH)),
            pltpu.SemaphoreType.DMA((2, RH)),
        ],
        compiler_params=pltpu.CompilerParams(
            collective_id=0, vmem_limit_bytes=96 * 1024 * 1024),
    )(jnp.asarray(_TBL), xt16, w16)


# device time: 45379 ns/iter; 1.9960x vs baseline; 1.1880x over previous
import jax
import jax.numpy as jnp
import numpy as np
from jax import lax
from jax.experimental import pallas as pl
from jax.experimental.pallas import tpu as pltpu

N = 16
RH = 8
LH = 7

_RING = np.array([0, 4, 8, 12, 15, 11, 7, 3, 2, 6, 10, 14, 13, 9, 5, 1])
_POS = np.argsort(_RING)

_M_RIGHT, _M_LEFT, _M_OWN, _M_RR, _M_RL = 0, 1, 2, 3, 11
_TBL = np.zeros((N, 18), np.int32)
for _l in range(N):
    _r = int(_POS[_l])
    _TBL[_l, _M_RIGHT] = _RING[(_r + 1) % N]
    _TBL[_l, _M_LEFT] = _RING[(_r - 1) % N]
    _TBL[_l, _M_OWN] = _l
    for _h in range(RH):
        _TBL[_l, _M_RR + _h] = _RING[(_r - _h - 1) % N]
    for _h in range(LH):
        _TBL[_l, _M_RL + _h] = _RING[(_r + _h + 1) % N]


def kernel(x, w_mat):
    xt16 = x.T.astype(jnp.bfloat16)
    w16 = w_mat.astype(jnp.bfloat16)
    kk, m = xt16.shape
    _, n = w16.shape

    def body(tbl_ref, xt_ref, w_ref, out_ref, xg, wg, ssx, ssw, rsx, rsw):
        my = lax.axis_index("i")
        right = tbl_ref[my, _M_RIGHT]
        left = tbl_ref[my, _M_LEFT]

        barrier = pltpu.get_barrier_semaphore()
        for nbr in (right, left):
            pl.semaphore_signal(barrier, inc=1, device_id=(nbr,),
                                device_id_type=pl.DeviceIdType.MESH)
        pl.semaphore_wait(barrier, 2)

        def rdma(src, dst, send_sem, recv_sem, dev):
            return pltpu.make_async_remote_copy(
                src_ref=src, dst_ref=dst, send_sem=send_sem,
                recv_sem=recv_sem, device_id=(dev,),
                device_id_type=pl.DeviceIdType.MESH)

        def gemm(xt_c, w_c):
            return lax.dot_general(
                xt_c, w_c, dimension_numbers=(((0,), (0,)), ((), ())),
                preferred_element_type=jnp.float32)

        own = tbl_ref[my, _M_OWN]
        sends = []
        for d, nbr in ((0, right), (1, left)):
            sx = rdma(xt_ref, xg.at[own], ssx.at[d, 0], rsx.at[d, 0], nbr)
            sw = rdma(w_ref, wg.at[own], ssw.at[d, 0], rsw.at[d, 0], nbr)
            sx.start()
            sw.start()
            sends += [sx, sw]

        acc = gemm(xt_ref[...], w_ref[...])
        for h in range(RH):
            ror = tbl_ref[my, _M_RR + h]
            rdma(xt_ref, xg.at[ror], ssx.at[0, h], rsx.at[0, h],
                 left).wait_recv()
            if h + 1 < RH:
                sx = rdma(xg.at[ror], xg.at[ror], ssx.at[0, h + 1],
                          rsx.at[0, h + 1], right)
                sx.start()
                sends.append(sx)
            rdma(w_ref, wg.at[ror], ssw.at[0, h], rsw.at[0, h],
                 left).wait_recv()
            if h + 1 < RH:
                sw = rdma(wg.at[ror], wg.at[ror], ssw.at[0, h + 1],
                          rsw.at[0, h + 1], right)
                sw.start()
                sends.append(sw)
            if h < LH:
                rol = tbl_ref[my, _M_RL + h]
                rdma(xt_ref, xg.at[rol], ssx.at[1, h], rsx.at[1, h],
                     right).wait_recv()
                if h + 1 < LH:
                    lx = rdma(xg.at[rol], xg.at[rol], ssx.at[1, h + 1],
                              rsx.at[1, h + 1], left)
                    lx.start()
                    sends.append(lx)
                rdma(w_ref, wg.at[rol], ssw.at[1, h], rsw.at[1, h],
                     right).wait_recv()
                if h + 1 < LH:
                    lw = rdma(wg.at[rol], wg.at[rol], ssw.at[1, h + 1],
                              rsw.at[1, h + 1], left)
                    lw.start()
                    sends.append(lw)

            acc += gemm(xg[ror], wg[ror])
            if h < LH:
                acc += gemm(xg[rol], wg[rol])

        out_ref[...] = jnp.maximum(acc, 0.0)
        for s in sends:
            s.wait_send()

    return pl.pallas_call(
        body,
        out_shape=jax.ShapeDtypeStruct((m, n), jnp.float32),
        in_specs=[
            pl.BlockSpec(memory_space=pltpu.SMEM),
            pl.BlockSpec(memory_space=pltpu.VMEM),
            pl.BlockSpec(memory_space=pltpu.VMEM),
        ],
        out_specs=pl.BlockSpec(memory_space=pltpu.VMEM),
        scratch_shapes=[
            pltpu.VMEM((N, kk, m), jnp.bfloat16),
            pltpu.VMEM((N, kk, n), jnp.bfloat16),
            pltpu.SemaphoreType.DMA((2, RH)),
            pltpu.SemaphoreType.DMA((2, RH)),
            pltpu.SemaphoreType.DMA((2, RH)),
            pltpu.SemaphoreType.DMA((2, RH)),
        ],
        compiler_params=pltpu.CompilerParams(
            collective_id=0, vmem_limit_bytes=96 * 1024 * 1024),
    )(jnp.asarray(_TBL), xt16, w16)


# device time: 45220 ns/iter; 2.0030x vs baseline; 1.0035x over previous
import jax
import jax.numpy as jnp
import numpy as np
from jax import lax
from jax.experimental import pallas as pl
from jax.experimental.pallas import tpu as pltpu

N = 16
RH = 8
LH = 7
NQ = 2
QROWS = 64 // NQ

_RING = np.array([0, 4, 8, 12, 15, 11, 7, 3, 2, 6, 10, 14, 13, 9, 5, 1])
_POS = np.argsort(_RING)

_M_RIGHT, _M_LEFT, _M_OWN, _M_RR, _M_RL = 0, 1, 2, 3, 11
_TBL = np.zeros((N, 18), np.int32)
for _l in range(N):
    _r = int(_POS[_l])
    _TBL[_l, _M_RIGHT] = _RING[(_r + 1) % N]
    _TBL[_l, _M_LEFT] = _RING[(_r - 1) % N]
    _TBL[_l, _M_OWN] = _l
    for _h in range(RH):
        _TBL[_l, _M_RR + _h] = _RING[(_r - _h - 1) % N]
    for _h in range(LH):
        _TBL[_l, _M_RL + _h] = _RING[(_r + _h + 1) % N]


def kernel(x, w_mat):
    xt16 = x.T.astype(jnp.bfloat16)
    w16 = w_mat.astype(jnp.bfloat16)
    kk, m = xt16.shape
    _, n = w16.shape

    def body(tbl_ref, xt_ref, w_ref, out_ref, xg, wg, ssx, ssw, rsx, rsw):
        my = lax.axis_index("i")
        right = tbl_ref[my, _M_RIGHT]
        left = tbl_ref[my, _M_LEFT]

        barrier = pltpu.get_barrier_semaphore()
        for nbr in (right, left):
            pl.semaphore_signal(barrier, inc=1, device_id=(nbr,),
                                device_id_type=pl.DeviceIdType.MESH)
        pl.semaphore_wait(barrier, 2)

        def rdma(src, dst, send_sem, recv_sem, dev):
            return pltpu.make_async_remote_copy(
                src_ref=src, dst_ref=dst, send_sem=send_sem,
                recv_sem=recv_sem, device_id=(dev,),
                device_id_type=pl.DeviceIdType.MESH)

        def gemm(xt_c, w_c):
            return lax.dot_general(
                xt_c, w_c, dimension_numbers=(((0,), (0,)), ((), ())),
                preferred_element_type=jnp.float32)

        streams = ((xt_ref, xg, ssx, rsx), (w_ref, wg, ssw, rsw))

        own = tbl_ref[my, _M_OWN]
        sends = []
        for d, nbr in ((0, right), (1, left)):
            for src, gbuf, ss, rs in streams:
                for q in range(NQ):
                    sl = pl.ds(q * QROWS, QROWS)
                    s = rdma(src.at[sl], gbuf.at[own, sl],
                             ss.at[d, 0, q], rs.at[d, 0, q], nbr)
                    s.start()
                    sends.append(s)

        acc = gemm(xt_ref[...], w_ref[...])
        for h in range(RH):
            ror = tbl_ref[my, _M_RR + h]
            for _, gbuf, ss, rs in streams:
                for q in range(NQ):
                    sl = pl.ds(q * QROWS, QROWS)
                    rdma(gbuf.at[ror, sl], gbuf.at[ror, sl],
                         ss.at[0, h, q], rs.at[0, h, q], left).wait_recv()
                    if h + 1 < RH:
                        s = rdma(gbuf.at[ror, sl], gbuf.at[ror, sl],
                                 ss.at[0, h + 1, q], rs.at[0, h + 1, q],
                                 right)
                        s.start()
                        sends.append(s)
            if h < LH:
                rol = tbl_ref[my, _M_RL + h]
                for _, gbuf, ss, rs in streams:
                    for q in range(NQ):
                        sl = pl.ds(q * QROWS, QROWS)
                        rdma(gbuf.at[rol, sl], gbuf.at[rol, sl],
                             ss.at[1, h, q], rs.at[1, h, q],
                             right).wait_recv()
                        if h + 1 < LH:
                            s = rdma(gbuf.at[rol, sl], gbuf.at[rol, sl],
                                     ss.at[1, h + 1, q], rs.at[1, h + 1, q],
                                     left)
                            s.start()
                            sends.append(s)

            acc += gemm(xg[ror], wg[ror])
            if h < LH:
                acc += gemm(xg[rol], wg[rol])

        out_ref[...] = jnp.maximum(acc, 0.0)
        for s in sends:
            s.wait_send()

    return pl.pallas_call(
        body,
        out_shape=jax.ShapeDtypeStruct((m, n), jnp.float32),
        in_specs=[
            pl.BlockSpec(memory_space=pltpu.SMEM),
            pl.BlockSpec(memory_space=pltpu.VMEM),
            pl.BlockSpec(memory_space=pltpu.VMEM),
        ],
        out_specs=pl.BlockSpec(memory_space=pltpu.VMEM),
        scratch_shapes=[
            pltpu.VMEM((N, kk, m), jnp.bfloat16),
            pltpu.VMEM((N, kk, n), jnp.bfloat16),
            pltpu.SemaphoreType.DMA((2, RH, NQ)),
            pltpu.SemaphoreType.DMA((2, RH, NQ)),
            pltpu.SemaphoreType.DMA((2, RH, NQ)),
            pltpu.SemaphoreType.DMA((2, RH, NQ)),
        ],
        compiler_params=pltpu.CompilerParams(
            collective_id=0, vmem_limit_bytes=96 * 1024 * 1024),
    )(jnp.asarray(_TBL), xt16, w16)


# device time: 41790 ns/iter; 2.1674x vs baseline; 1.0821x over previous
import jax
import jax.numpy as jnp
import numpy as np
from jax import lax
from jax.experimental import pallas as pl
from jax.experimental.pallas import tpu as pltpu

N = 16
KC = 64
RH = 8
LH = 7
NQ = 2
QROWS = KC // NQ

_RING = np.array([0, 4, 8, 12, 15, 11, 7, 3, 2, 6, 10, 14, 13, 9, 5, 1])
_POS = np.argsort(_RING)

_M_RIGHT, _M_LEFT, _M_OWN, _M_RR, _M_RL = 0, 1, 2, 3, 11
_TBL = np.zeros((N, 18), np.int32)
for _l in range(N):
    _r = int(_POS[_l])
    _TBL[_l, _M_RIGHT] = _RING[(_r + 1) % N]
    _TBL[_l, _M_LEFT] = _RING[(_r - 1) % N]
    _TBL[_l, _M_OWN] = _l
    for _h in range(RH):
        _TBL[_l, _M_RR + _h] = _RING[(_r - _h - 1) % N]
    for _h in range(LH):
        _TBL[_l, _M_RL + _h] = _RING[(_r + _h + 1) % N]


def kernel(x, w_mat):
    xt16 = x.T.astype(jnp.bfloat16)
    w16 = w_mat.astype(jnp.bfloat16)
    kk, m = xt16.shape
    _, n = w16.shape

    def body(tbl_ref, xt_ref, w_ref, out_ref, xg, wg, ssx, ssw, rsx, rsw,
             lsem):
        my = lax.axis_index("i")
        right = tbl_ref[my, _M_RIGHT]
        left = tbl_ref[my, _M_LEFT]

        barrier = pltpu.get_barrier_semaphore()
        for nbr in (right, left):
            pl.semaphore_signal(barrier, inc=1, device_id=(nbr,),
                                device_id_type=pl.DeviceIdType.MESH)
        pl.semaphore_wait(barrier, 2)

        def rdma(src, dst, send_sem, recv_sem, dev):
            return pltpu.make_async_remote_copy(
                src_ref=src, dst_ref=dst, send_sem=send_sem,
                recv_sem=recv_sem, device_id=(dev,),
                device_id_type=pl.DeviceIdType.MESH)

        streams = ((xt_ref, xg, ssx, rsx), (w_ref, wg, ssw, rsw))

        own = tbl_ref[my, _M_OWN]
        own_rows = pl.ds(own * KC, KC)
        sends = []
        for d, nbr in ((0, right), (1, left)):
            for src, gbuf, ss, rs in streams:
                for q in range(NQ):
                    s = rdma(src.at[pl.ds(q * QROWS, QROWS)],
                             gbuf.at[pl.ds(own * KC + q * QROWS, QROWS)],
                             ss.at[d, 0, q], rs.at[d, 0, q], nbr)
                    s.start()
                    sends.append(s)
        cpx = pltpu.make_async_copy(xt_ref, xg.at[own_rows], lsem.at[0])
        cpw = pltpu.make_async_copy(w_ref, wg.at[own_rows], lsem.at[1])
        cpx.start()
        cpw.start()

        for h in range(RH):
            ror = tbl_ref[my, _M_RR + h]
            for _, gbuf, ss, rs in streams:
                for q in range(NQ):
                    sl = pl.ds(ror * KC + q * QROWS, QROWS)
                    rdma(gbuf.at[sl], gbuf.at[sl],
                         ss.at[0, h, q], rs.at[0, h, q], left).wait_recv()
                    if h + 1 < RH:
                        s = rdma(gbuf.at[sl], gbuf.at[sl],
                                 ss.at[0, h + 1, q], rs.at[0, h + 1, q],
                                 right)
                        s.start()
                        sends.append(s)
            if h < LH:
                rol = tbl_ref[my, _M_RL + h]
                for _, gbuf, ss, rs in streams:
                    for q in range(NQ):
                        sl = pl.ds(rol * KC + q * QROWS, QROWS)
                        rdma(gbuf.at[sl], gbuf.at[sl],
                             ss.at[1, h, q], rs.at[1, h, q],
                             right).wait_recv()
                        if h + 1 < LH:
                            s = rdma(gbuf.at[sl], gbuf.at[sl],
                                     ss.at[1, h + 1, q], rs.at[1, h + 1, q],
                                     left)
                            s.start()
                            sends.append(s)

        cpx.wait()
        cpw.wait()
        acc = lax.dot_general(xg[...], wg[...],
                              dimension_numbers=(((0,), (0,)), ((), ())),
                              preferred_element_type=jnp.float32)
        out_ref[...] = jnp.maximum(acc, 0.0)
        for s in sends:
            s.wait_send()

    return pl.pallas_call(
        body,
        out_shape=jax.ShapeDtypeStruct((m, n), jnp.float32),
        in_specs=[
            pl.BlockSpec(memory_space=pltpu.SMEM),
            pl.BlockSpec(memory_space=pltpu.VMEM),
            pl.BlockSpec(memory_space=pltpu.VMEM),
        ],
        out_specs=pl.BlockSpec(memory_space=pltpu.VMEM),
        scratch_shapes=[
            pltpu.VMEM((N * KC, m), jnp.bfloat16),
            pltpu.VMEM((N * KC, n), jnp.bfloat16),
            pltpu.SemaphoreType.DMA((2, RH, NQ)),
            pltpu.SemaphoreType.DMA((2, RH, NQ)),
            pltpu.SemaphoreType.DMA((2, RH, NQ)),
            pltpu.SemaphoreType.DMA((2, RH, NQ)),
            pltpu.SemaphoreType.DMA((2,)),
        ],
        compiler_params=pltpu.CompilerParams(
            collective_id=0, vmem_limit_bytes=96 * 1024 * 1024),
    )(jnp.asarray(_TBL), xt16, w16)


# device time: 39666 ns/iter; 2.2835x vs baseline; 1.0535x over previous
import jax
import jax.numpy as jnp
import numpy as np
from jax import lax
from jax.experimental import pallas as pl
from jax.experimental.pallas import tpu as pltpu

N = 16
KC = 64
RH = 8
LH = 7
NQ = 2
QROWS = KC // NQ
NSLOT = RH + LH

_RING = np.array([0, 4, 8, 12, 15, 11, 7, 3, 2, 6, 10, 14, 13, 9, 5, 1])
_POS = np.argsort(_RING)

_TBL = np.zeros((N, 2), np.int32)
for _l in range(N):
    _r = int(_POS[_l])
    _TBL[_l, 0] = _RING[(_r + 1) % N]
    _TBL[_l, 1] = _RING[(_r - 1) % N]


def _rslot(h):
    return 2 * h


def _lslot(h):
    return 2 * h + 1


def kernel(x, w_mat):
    xt16 = x.T.astype(jnp.bfloat16)
    kk, m = xt16.shape
    n = w_mat.shape[1]
    nh = n // 2

    def body(tbl_ref, xt_ref, wf_ref, out_ref, xg, wg, w16, obuf, ssx, ssw,
             rsx, rsw, osem):
        my = lax.axis_index("i")
        right = tbl_ref[my, 0]
        left = tbl_ref[my, 1]

        barrier = pltpu.get_barrier_semaphore()
        for nbr in (right, left):
            pl.semaphore_signal(barrier, inc=1, device_id=(nbr,),
                                device_id_type=pl.DeviceIdType.MESH)
        pl.semaphore_wait(barrier, 2)

        def rdma(src, dst, send_sem, recv_sem, dev):
            return pltpu.make_async_remote_copy(
                src_ref=src, dst_ref=dst, send_sem=send_sem,
                recv_sem=recv_sem, device_id=(dev,),
                device_id_type=pl.DeviceIdType.MESH)

        w16[...] = wf_ref[...].astype(jnp.bfloat16)
        streams = ((xt_ref, xg, ssx, rsx), (w16, wg, ssw, rsw))

        def qrows(slot, q):
            return pl.ds(slot * KC + q * QROWS, QROWS)

        sends = []
        for d, nbr, slot in ((0, right, _rslot(0)), (1, left, _lslot(0))):
            for src, gbuf, ss, rs in streams:
                for q in range(NQ):
                    s = rdma(src.at[pl.ds(q * QROWS, QROWS)],
                             gbuf.at[qrows(slot, q)],
                             ss.at[d, 0, q], rs.at[d, 0, q], nbr)
                    s.start()
                    sends.append(s)

        def tdot(a, b):
            return lax.dot_general(
                a, b, dimension_numbers=(((0,), (0,)), ((), ())),
                preferred_element_type=jnp.float32)

        accs = [tdot(xt_ref[...], w16[:, i * nh:(i + 1) * nh])
                for i in range(2)]

        for h in range(RH):
            for _, gbuf, ss, rs in streams:
                for q in range(NQ):
                    sl = qrows(_rslot(h), q)
                    rdma(gbuf.at[sl], gbuf.at[sl],
                         ss.at[0, h, q], rs.at[0, h, q], left).wait_recv()
                    if h + 1 < RH:
                        s = rdma(gbuf.at[sl], gbuf.at[qrows(_rslot(h + 1), q)],
                                 ss.at[0, h + 1, q], rs.at[0, h + 1, q],
                                 right)
                        s.start()
                        sends.append(s)
                    if h < LH:
                        sl = qrows(_lslot(h), q)
                        rdma(gbuf.at[sl], gbuf.at[sl],
                             ss.at[1, h, q], rs.at[1, h, q],
                             right).wait_recv()
                        if h + 1 < LH:
                            s = rdma(gbuf.at[sl],
                                     gbuf.at[qrows(_lslot(h + 1), q)],
                                     ss.at[1, h + 1, q], rs.at[1, h + 1, q],
                                     left)
                            s.start()
                            sends.append(s)

            lo = _rslot(h) * KC
            nrows = 2 * KC if h < LH else KC
            for i in range(2):
                accs[i] += tdot(xg[lo:lo + nrows, :],
                                wg[lo:lo + nrows, i * nh:(i + 1) * nh])

        ocopies = []
        for i in range(2):
            obuf[i] = jnp.maximum(accs[i], 0.0)
            oc = pltpu.make_async_copy(
                obuf.at[i], out_ref.at[:, pl.ds(i * nh, nh)], osem.at[i])
            oc.start()
            ocopies.append(oc)
        for oc in ocopies:
            oc.wait()
        for s in sends:
            s.wait_send()

    return pl.pallas_call(
        body,
        out_shape=jax.ShapeDtypeStruct((m, n), jnp.float32),
        in_specs=[
            pl.BlockSpec(memory_space=pltpu.SMEM),
            pl.BlockSpec(memory_space=pltpu.VMEM),
            pl.BlockSpec(memory_space=pltpu.VMEM),
        ],
        out_specs=pl.BlockSpec(memory_space=pltpu.MemorySpace.HBM),
        scratch_shapes=[
            pltpu.VMEM((NSLOT * KC, m), jnp.bfloat16),
            pltpu.VMEM((NSLOT * KC, n), jnp.bfloat16),
            pltpu.VMEM((KC, n), jnp.bfloat16),
            pltpu.VMEM((2, m, n // 2), jnp.float32),
            pltpu.SemaphoreType.DMA((2, RH, NQ)),
            pltpu.SemaphoreType.DMA((2, RH, NQ)),
            pltpu.SemaphoreType.DMA((2, RH, NQ)),
            pltpu.SemaphoreType.DMA((2, RH, NQ)),
            pltpu.SemaphoreType.DMA((2,)),
        ],
        compiler_params=pltpu.CompilerParams(
            collective_id=0, vmem_limit_bytes=96 * 1024 * 1024),
    )(jnp.asarray(_TBL), xt16, w_mat)
